# Initial kernel scaffold; baseline (speedup 1.0000x reference)
import jax
import jax.numpy as jnp
from jax.experimental import pallas as pl


def kernel(node_features, edge_index, edge_features, W, b, gamma, beta):
    raise NotImplementedError("write your pallas kernel here")


# speedup vs baseline: 2.3865x; 2.3865x over previous
import functools

import jax
import jax.numpy as jnp
from jax import lax
from jax.experimental import pallas as pl
from jax.experimental.pallas import tpu as pltpu
from jax.experimental.pallas import tpu_sc as plsc

N_NODES = 10000
N_EDGES = 320000
D_FEAT = 128
D_EDGE = 16
UNITS = 128
EPS = 1e-3

NC = 2
NS = 16
NW = NC * NS
EDGES_PER_WORKER = N_EDGES // NW
CHUNK = 80
N_CHUNKS = EDGES_PER_WORKER // CHUNK
N_PAD = 10240
ROWS_PER_TILE = N_PAD // NS
ZROWS = 128


def _proj_nodes_body(nf_ref, w_ref, out_ref):
    out_ref[...] = jnp.dot(nf_ref[...], w_ref[...],
                           preferred_element_type=jnp.float32)


def _proj_edges_body(ef_ref, w_ref, b_ref, out_ref):
    out_ref[...] = jnp.dot(ef_ref[...], w_ref[...],
                           preferred_element_type=jnp.float32) + b_ref[...]


def _project(node_features, edge_features, w_node, w_edge, b2d):
    p = pl.pallas_call(
        _proj_nodes_body,
        grid=(5,),
        in_specs=[
            pl.BlockSpec((2000, D_FEAT), lambda i: (i, 0)),
            pl.BlockSpec((D_FEAT, UNITS), lambda i: (0, 0)),
        ],
        out_specs=pl.BlockSpec((2000, UNITS), lambda i: (i, 0)),
        out_shape=jax.ShapeDtypeStruct((N_NODES, UNITS), jnp.float32),
    )(node_features, w_node)

    e = pl.pallas_call(
        _proj_edges_body,
        grid=(40,),
        in_specs=[
            pl.BlockSpec((8000, D_EDGE), lambda i: (i, 0)),
            pl.BlockSpec((D_EDGE, UNITS), lambda i: (0, 0)),
            pl.BlockSpec((1, UNITS), lambda i: (0, 0)),
        ],
        out_specs=pl.BlockSpec((8000, UNITS), lambda i: (i, 0)),
        out_shape=jax.ShapeDtypeStruct((N_EDGES, UNITS), jnp.float32),
    )(edge_features, w_edge, b2d)
    return p, e


def _sc_body(p_hbm, e_hbm, src_hbm, tgt_hbm, part_hbm,
             acc, idx_s, idx_t, g_v, e_v, zbuf, sem):
    cid = lax.axis_index("c")
    sid = lax.axis_index("s")
    wid = sid * NC + cid

    def _zrow(i, _):
        r = i // 8
        j = i % 8
        zbuf[r, pl.ds(j * 16, 16)] = jnp.zeros((16,), jnp.float32)
        return 0
    lax.fori_loop(0, ZROWS * 8, _zrow, 0)
    for k in range(ROWS_PER_TILE // ZROWS):
        pltpu.sync_copy(zbuf, acc.at[pl.ds(sid * ROWS_PER_TILE + k * ZROWS,
                                           ZROWS)])
    plsc.subcore_barrier()

    base = wid * EDGES_PER_WORKER

    def _chunk(c, _):
        off = base + c * CHUNK
        pltpu.sync_copy(src_hbm.at[pl.ds(off, CHUNK)], idx_s)
        pltpu.sync_copy(tgt_hbm.at[pl.ds(off, CHUNK)], idx_t)
        pltpu.async_copy(p_hbm.at[idx_s], g_v, sem).wait()
        pltpu.sync_copy(e_hbm.at[pl.ds(off, CHUNK)], e_v)

        def _row(r, _):
            def _col(j, _):
                x = g_v[r, pl.ds(j * 16, 16)] + e_v[r, pl.ds(j * 16, 16)]
                y = x / (1.0 + jnp.exp(-x))
                g_v[r, pl.ds(j * 16, 16)] = y
                return 0
            lax.fori_loop(0, 8, _col, 0)
            return 0
        lax.fori_loop(0, CHUNK, _row, 0)

        pltpu.sync_copy(g_v, acc.at[idx_t], add=True)
        return 0
    lax.fori_loop(0, N_CHUNKS, _chunk, 0)

    plsc.subcore_barrier()
    for k in range(ROWS_PER_TILE // ZROWS):
        r0 = sid * ROWS_PER_TILE + k * ZROWS
        pltpu.sync_copy(acc.at[pl.ds(r0, ZROWS)],
                        part_hbm.at[cid, pl.ds(r0, ZROWS)])


def _sc_aggregate(p, e, src, tgt):
    mesh = plsc.VectorSubcoreMesh(core_axis_name="c", subcore_axis_name="s")
    f = pl.kernel(
        _sc_body,
        out_type=jax.ShapeDtypeStruct((NC, N_PAD, UNITS), jnp.float32),
        mesh=mesh,
        scratch_types=[
            pltpu.VMEM_SHARED((N_PAD, UNITS), jnp.float32),
            pltpu.VMEM((CHUNK,), jnp.int32),
            pltpu.VMEM((CHUNK,), jnp.int32),
            pltpu.VMEM((CHUNK, UNITS), jnp.float32),
            pltpu.VMEM((CHUNK, UNITS), jnp.float32),
            pltpu.VMEM((ZROWS, UNITS), jnp.float32),
            pltpu.SemaphoreType.DMA,
        ],
    )
    return f(p, e, src, tgt)


def _ln_body(part_ref, g_ref, b_ref, out_ref):
    s = part_ref[0] + part_ref[1]
    mean = jnp.mean(s, axis=-1, keepdims=True)
    var = jnp.mean(jnp.square(s - mean), axis=-1, keepdims=True)
    out_ref[...] = (s - mean) * lax.rsqrt(var + EPS) * g_ref[...] + b_ref[...]


def _layernorm(partials, gamma2d, beta2d):
    return pl.pallas_call(
        _ln_body,
        grid=(5,),
        in_specs=[
            pl.BlockSpec((NC, 2000, UNITS), lambda i: (0, i, 0)),
            pl.BlockSpec((1, UNITS), lambda i: (0, 0)),
            pl.BlockSpec((1, UNITS), lambda i: (0, 0)),
        ],
        out_specs=pl.BlockSpec((2000, UNITS), lambda i: (i, 0)),
        out_shape=jax.ShapeDtypeStruct((N_NODES, UNITS), jnp.float32),
    )(partials, gamma2d, beta2d)


@jax.jit
def kernel(node_features, edge_index, edge_features, W, b, gamma, beta):
    src = edge_index[0].astype(jnp.int32)
    tgt = edge_index[1].astype(jnp.int32)
    w_node = W[:D_FEAT]
    w_edge = W[D_FEAT:]
    p, e = _project(node_features, edge_features, w_node, w_edge,
                    b.reshape(1, UNITS))
    partials = _sc_aggregate(p, e, src, tgt)
    return _layernorm(partials, gamma.reshape(1, UNITS),
                      beta.reshape(1, UNITS))


# speedup vs baseline: 3.6673x; 1.5367x over previous
import functools

import jax
import jax.numpy as jnp
from jax import lax
from jax.experimental import pallas as pl
from jax.experimental.pallas import tpu as pltpu
from jax.experimental.pallas import tpu_sc as plsc

N_NODES = 10000
N_EDGES = 320000
D_FEAT = 128
D_EDGE = 16
UNITS = 128
EPS = 1e-3

NC = 2
NS = 16
NW = NC * NS
EDGES_PER_WORKER = N_EDGES // NW
CHUNK = 80
N_CHUNKS = EDGES_PER_WORKER // CHUNK
N_PAD = 10240
ROWS_PER_TILE = N_PAD // NS
ZROWS = 128


def _proj_nodes_body(nf_ref, w_ref, out_ref):
    out_ref[...] = jnp.dot(nf_ref[...], w_ref[...],
                           preferred_element_type=jnp.float32)


def _proj_edges_body(ef_ref, w_ref, b_ref, out_ref):
    out_ref[...] = jnp.dot(ef_ref[...], w_ref[...],
                           preferred_element_type=jnp.float32) + b_ref[...]


def _project(node_features, edge_features, w_node, w_edge, b2d):
    p = pl.pallas_call(
        _proj_nodes_body,
        grid=(5,),
        in_specs=[
            pl.BlockSpec((2000, D_FEAT), lambda i: (i, 0)),
            pl.BlockSpec((D_FEAT, UNITS), lambda i: (0, 0)),
        ],
        out_specs=pl.BlockSpec((2000, UNITS), lambda i: (i, 0)),
        out_shape=jax.ShapeDtypeStruct((N_NODES, UNITS), jnp.float32),
    )(node_features, w_node)

    e = pl.pallas_call(
        _proj_edges_body,
        grid=(40,),
        in_specs=[
            pl.BlockSpec((8000, D_EDGE), lambda i: (i, 0)),
            pl.BlockSpec((D_EDGE, UNITS), lambda i: (0, 0)),
            pl.BlockSpec((1, UNITS), lambda i: (0, 0)),
        ],
        out_specs=pl.BlockSpec((8000, UNITS), lambda i: (i, 0)),
        out_shape=jax.ShapeDtypeStruct((N_EDGES, UNITS), jnp.float32),
    )(edge_features, w_edge, b2d)
    return p, e


def _silu_chunk(g_v, e_v):
    def _row(r, _):
        for j in range(8):
            x = g_v[r, pl.ds(j * 16, 16)] + e_v[r, pl.ds(j * 16, 16)]
            e_v[r, pl.ds(j * 16, 16)] = x / (1.0 + jnp.exp(-x))
        return 0
    lax.fori_loop(0, CHUNK, _row, 0)


def _sc_body(p_hbm, e_hbm, idx3_hbm, part_hbm,
             acc, i0, i1, g0, g1, e0, e1,
             gsem0, gsem1, esem0, esem1):
    cid = lax.axis_index("c")
    sid = lax.axis_index("s")
    wid = sid * NC + cid
    ibuf = (i0, i1)
    gbuf = (g0, g1)
    ebuf = (e0, e1)
    gsem = (gsem0, gsem1)
    esem = (esem0, esem1)

    def _zrow(r, _):
        for j in range(8):
            g0[r, pl.ds(j * 16, 16)] = jnp.zeros((16,), jnp.float32)
        return 0
    lax.fori_loop(0, CHUNK, _zrow, 0)
    for k in range(ROWS_PER_TILE // CHUNK):
        pltpu.sync_copy(g0, acc.at[pl.ds(sid * ROWS_PER_TILE + k * CHUNK,
                                         CHUNK)])
    plsc.subcore_barrier()

    base = wid * EDGES_PER_WORKER

    def _start(c, b):
        pltpu.sync_copy(idx3_hbm.at[wid * N_CHUNKS + c], ibuf[b])
        pltpu.async_copy(p_hbm.at[ibuf[b].at[0]], gbuf[b], gsem[b])
        pltpu.async_copy(e_hbm.at[pl.ds(base + c * CHUNK, CHUNK)],
                         ebuf[b], esem[b])

    def _finish(c, b, prefetch_c):
        pltpu.make_async_copy(p_hbm.at[ibuf[b].at[0]], gbuf[b],
                              gsem[b]).wait()
        pltpu.make_async_copy(e_hbm.at[pl.ds(base + c * CHUNK, CHUNK)],
                              ebuf[b], esem[b]).wait()
        _silu_chunk(gbuf[b], ebuf[b])
        pltpu.sync_copy(ebuf[b], acc.at[ibuf[b].at[1]], add=True)
        if prefetch_c is not None:
            @pl.when(prefetch_c < N_CHUNKS)
            def _():
                _start(prefetch_c, b)

    _start(0, 0)
    _start(1, 1)

    def _pair(i, _):
        c = 2 * i
        _finish(c, 0, c + 2)
        _finish(c + 1, 1, c + 3)
        return 0
    lax.fori_loop(0, N_CHUNKS // 2, _pair, 0)
    _finish(N_CHUNKS - 1, 0, None)

    plsc.subcore_barrier()
    for k in range(ROWS_PER_TILE // CHUNK):
        r0 = sid * ROWS_PER_TILE + k * CHUNK
        pltpu.sync_copy(acc.at[pl.ds(r0, CHUNK)],
                        part_hbm.at[cid, pl.ds(r0, CHUNK)])


def _sc_aggregate(p, e, idx3):
    mesh = plsc.VectorSubcoreMesh(core_axis_name="c", subcore_axis_name="s")
    f = pl.kernel(
        _sc_body,
        out_type=jax.ShapeDtypeStruct((NC, N_PAD, UNITS), jnp.float32),
        mesh=mesh,
        scratch_types=[
            pltpu.VMEM_SHARED((N_PAD, UNITS), jnp.float32),
            pltpu.VMEM((2, CHUNK), jnp.int32),
            pltpu.VMEM((2, CHUNK), jnp.int32),
            pltpu.VMEM((CHUNK, UNITS), jnp.float32),
            pltpu.VMEM((CHUNK, UNITS), jnp.float32),
            pltpu.VMEM((CHUNK, UNITS), jnp.float32),
            pltpu.VMEM((CHUNK, UNITS), jnp.float32),
            pltpu.SemaphoreType.DMA,
            pltpu.SemaphoreType.DMA,
            pltpu.SemaphoreType.DMA,
            pltpu.SemaphoreType.DMA,
        ],
    )
    return f(p, e, idx3)


def _ln_body(part_ref, g_ref, b_ref, out_ref):
    s = part_ref[0] + part_ref[1]
    mean = jnp.mean(s, axis=-1, keepdims=True)
    var = jnp.mean(jnp.square(s - mean), axis=-1, keepdims=True)
    out_ref[...] = (s - mean) * lax.rsqrt(var + EPS) * g_ref[...] + b_ref[...]


def _layernorm(partials, gamma2d, beta2d):
    return pl.pallas_call(
        _ln_body,
        grid=(5,),
        in_specs=[
            pl.BlockSpec((NC, 2000, UNITS), lambda i: (0, i, 0)),
            pl.BlockSpec((1, UNITS), lambda i: (0, 0)),
            pl.BlockSpec((1, UNITS), lambda i: (0, 0)),
        ],
        out_specs=pl.BlockSpec((2000, UNITS), lambda i: (i, 0)),
        out_shape=jax.ShapeDtypeStruct((N_NODES, UNITS), jnp.float32),
    )(partials, gamma2d, beta2d)


@jax.jit
def kernel(node_features, edge_index, edge_features, W, b, gamma, beta):
    idx3 = (edge_index.astype(jnp.int32)
            .reshape(2, NW, N_CHUNKS, CHUNK)
            .transpose(1, 2, 0, 3)
            .reshape(NW * N_CHUNKS, 2, CHUNK))
    w_node = W[:D_FEAT]
    w_edge = W[D_FEAT:]
    p, e = _project(node_features, edge_features, w_node, w_edge,
                    b.reshape(1, UNITS))
    partials = _sc_aggregate(p, e, idx3)
    return _layernorm(partials, gamma.reshape(1, UNITS),
                      beta.reshape(1, UNITS))
